# baseline (speedup 1.0000x reference)
import functools

import jax
import jax.numpy as jnp
from jax import lax
from jax.experimental import pallas as pl
from jax.experimental.pallas import tpu as pltpu, tpu_sc as plsc

_D = 10
_K = 32
_CH = 1024

_GDN = lax.GatherDimensionNumbers(
    offset_dims=(), collapsed_slice_dims=(0,), start_index_map=(0,))


def _shuffle(v, idx):
    return lax.gather(v, idx[:, None], _GDN, (1,),
                      mode=lax.GatherScatterMode.PROMISE_IN_BOUNDS)


def _make_sc_decode(total_rows):
    info = plsc.get_sparse_core_info()
    nw = info.num_cores * info.num_subcores
    rows_w = total_rows // nw
    nch = rows_w // _CH
    mesh = plsc.VectorSubcoreMesh(core_axis_name="c", subcore_axis_name="s")

    @functools.partial(
        pl.kernel,
        mesh=mesh,
        out_type=jax.ShapeDtypeStruct((total_rows * _D,), jnp.float32),
        scratch_types=[
            pltpu.VMEM((_CH * _D + 16,), jnp.float32),
            pltpu.VMEM((_CH * _D + 16,), jnp.float32),
            pltpu.VMEM((_K * _D + 16,), jnp.float32),
            pltpu.VMEM((_K * _D + 16,), jnp.float32),
            pltpu.VMEM((_K,), jnp.float32),
        ],
    )
    def sc_decode(sig_hbm, cbt_hbm, cbr_hbm, nh_hbm, out_hbm,
                  in_v, out_v, cbt_v, cbr_v, nh_v):
        wid = lax.axis_index("s") * info.num_cores + lax.axis_index("c")
        pltpu.sync_copy(cbt_hbm, cbt_v.at[pl.ds(0, _K * _D)])
        pltpu.sync_copy(cbr_hbm, cbr_v.at[pl.ds(0, _K * _D)])
        pltpu.sync_copy(nh_hbm, nh_v)
        cb_lo = [cbt_v[pl.ds(d * _K, 16)] for d in range(_D)]
        cb_hi = [cbt_v[pl.ds(d * _K + 16, 16)] for d in range(_D)]
        nh_lo = nh_v[pl.ds(0, 16)]
        nh_hi = nh_v[pl.ds(16, 16)]
        iota = lax.iota(jnp.int32, 16)
        lane_mask = iota < _D
        base_w = wid * (rows_w * _D)

        def chunk_body(c, _):
            base = base_w + c * (_CH * _D)
            pltpu.sync_copy(sig_hbm.at[pl.ds(base, _CH * _D)],
                            in_v.at[pl.ds(0, _CH * _D)])

            def row_body(r, _):
                o = r * _D
                xr = in_v[pl.ds(o, 16)]
                t = xr * 65537.0
                xv = t - (t - xr)
                lo = nh_lo + xv[0] * cb_lo[0]
                hi = nh_hi + xv[0] * cb_hi[0]
                for d in range(1, _D):
                    xd = xv[d]
                    lo = lo + xd * cb_lo[d]
                    hi = hi + xd * cb_hi[d]
                m = jnp.maximum(lo, hi)
                for s in (8, 4, 2, 1):
                    m = jnp.maximum(m, _shuffle(m, iota ^ s))
                k = jnp.minimum(jnp.where(lo == m, iota, 64),
                                jnp.where(hi == m, iota + 16, 64))
                for s in (8, 4, 2, 1):
                    k = jnp.minimum(k, _shuffle(k, iota ^ s))
                out_v[pl.ds(r * _D, 16)] = cbr_v[pl.ds(k[0] * _D, 16)]
                return _

            lax.fori_loop(0, _CH, row_body, 0, unroll=4)
            pltpu.sync_copy(out_v.at[pl.ds(0, _CH * _D)],
                            out_hbm.at[pl.ds(base, _CH * _D)])
            return _

        lax.fori_loop(0, nch, chunk_body, 0)

    return sc_decode


def kernel(signal, codebook):
    b, n, d = signal.shape
    total_rows = b * n
    sig_flat = signal.reshape(total_rows * d)
    cbt = codebook.T.reshape(-1)
    cbr = codebook.reshape(-1)
    nh = -0.5 * jnp.sum(codebook * codebook, axis=1)
    out_flat = _make_sc_decode(total_rows)(sig_flat, cbt, cbr, nh)
    return out_flat.reshape(b, n, d)
